# baseline (speedup 1.0000x reference)
import functools

import jax
import jax.numpy as jnp
from jax import lax
from jax.experimental import pallas as pl
from jax.experimental.pallas import tpu as pltpu
from jax.experimental.pallas import tpu_sc as plsc

DIM = 32
MAX_REL_POS = 128
LENGTH_Q = 2048
LENGTH_K = 2048

_NUM_CORES = 2
_NUM_SUBCORES = 16
_NUM_TILES = _NUM_CORES * _NUM_SUBCORES
_NPHASE = 128
_S_COLS = 4352
_ROWS_PER_TILE = LENGTH_Q // _NUM_TILES
_BATCH = 8


def _stage_body(table_t_ref, out_ref):
    c = pl.program_id(0)
    upos = lax.broadcasted_iota(jnp.int32, (MAX_REL_POS * 2 + 1, _S_COLS), 1)
    idx = jnp.clip(
        upos + (c - (LENGTH_Q - 1)), -MAX_REL_POS, MAX_REL_POS
    ) + MAX_REL_POS
    rows = lax.broadcasted_iota(jnp.int32, (MAX_REL_POS * 2 + 1, _S_COLS), 0)
    onehot = (idx == rows).astype(jnp.float32)
    out_ref[0] = jnp.dot(
        table_t_ref[...], onehot, preferred_element_type=jnp.float32
    )


def _copy_body(st128_hbm, out_hbm, sem):
    c = lax.axis_index("c")
    s = lax.axis_index("s")
    g = c * _NUM_SUBCORES + s
    row0 = g * _ROWS_PER_TILE

    def copy_batch(b, _):
        i0 = row0 + b * _BATCH
        handles = []
        for t in range(_BATCH):
            i = i0 + t
            u0 = (LENGTH_Q - 1) - i
            ph = lax.rem(u0, _NPHASE)
            q = lax.div(u0, _NPHASE)
            src = st128_hbm.at[
                ph, :, pl.ds(pl.multiple_of(q * _NPHASE, _NPHASE), LENGTH_K)
            ]
            handles.append(pltpu.async_copy(src, out_hbm.at[i], sem))
        for h in handles:
            h.wait()
        return _

    lax.fori_loop(0, _ROWS_PER_TILE // _BATCH, copy_batch, None)


@jax.jit
def _rpe(table):
    table_t = table.T

    st128 = pl.pallas_call(
        _stage_body,
        grid=(_NPHASE,),
        in_specs=[
            pl.BlockSpec((DIM, MAX_REL_POS * 2 + 1), lambda c: (0, 0)),
        ],
        out_specs=pl.BlockSpec((1, DIM, _S_COLS), lambda c: (c, 0, 0)),
        out_shape=jax.ShapeDtypeStruct((_NPHASE, DIM, _S_COLS), jnp.float32),
    )(table_t)

    mesh = plsc.VectorSubcoreMesh(core_axis_name="c", subcore_axis_name="s")
    out3 = pl.kernel(
        _copy_body,
        out_type=jax.ShapeDtypeStruct((LENGTH_Q, DIM, LENGTH_K), jnp.float32),
        mesh=mesh,
        scratch_types=[pltpu.SemaphoreType.DMA],
    )(st128)
    return jnp.transpose(out3, (0, 2, 1))


def kernel(length_q, length_k, rel_pos_embeddings):
    del length_q, length_k
    return _rpe(rel_pos_embeddings)


# speedup vs baseline: 55.4576x; 55.4576x over previous
import functools

import jax
import jax.numpy as jnp
from jax import lax
from jax.experimental import pallas as pl
from jax.experimental.pallas import tpu as pltpu
from jax.experimental.pallas import tpu_sc as plsc

DIM = 32
MAX_REL_POS = 128
LENGTH_Q = 2048
LENGTH_K = 2048

_NUM_CORES = 2
_NUM_SUBCORES = 16
_NUM_TILES = _NUM_CORES * _NUM_SUBCORES
_NPHASE = 128
_NQ = 16
_S_COLS = _NPHASE * (_NQ - 1) + LENGTH_K
_PHASES_PER_TILE = _NPHASE // _NUM_TILES


def _stage_body(table_t_ref, out_ref):
    c = pl.program_id(0)
    upos = lax.broadcasted_iota(jnp.int32, (MAX_REL_POS * 2 + 1, _S_COLS), 1)
    idx = jnp.clip(
        upos + (c - (LENGTH_Q - 1)), -MAX_REL_POS, MAX_REL_POS
    ) + MAX_REL_POS
    rows = lax.broadcasted_iota(jnp.int32, (MAX_REL_POS * 2 + 1, _S_COLS), 0)
    onehot = (idx == rows).astype(jnp.float32)
    out_ref[0] = jnp.dot(
        table_t_ref[...], onehot, preferred_element_type=jnp.float32
    )


def _copy_body(st128_hbm, out_hbm, win, sem):
    c = lax.axis_index("c")
    s = lax.axis_index("s")
    g = c * _NUM_SUBCORES + s

    def do_phase(k, _):
        p = g * _PHASES_PER_TILE + k
        pltpu.sync_copy(st128_hbm.at[p], win)
        handles = []
        for q in range(_NQ):
            i = (LENGTH_Q - 1) - p - q * _NPHASE
            src = win.at[:, pl.ds(q * _NPHASE, LENGTH_K)]
            handles.append(pltpu.async_copy(src, out_hbm.at[i], sem))
        for h in handles:
            h.wait()
        return _

    lax.fori_loop(0, _PHASES_PER_TILE, do_phase, None)


@jax.jit
def _rpe(table):
    table_t = table.T

    st128 = pl.pallas_call(
        _stage_body,
        grid=(_NPHASE,),
        in_specs=[
            pl.BlockSpec((DIM, MAX_REL_POS * 2 + 1), lambda c: (0, 0)),
        ],
        out_specs=pl.BlockSpec((1, DIM, _S_COLS), lambda c: (c, 0, 0)),
        out_shape=jax.ShapeDtypeStruct((_NPHASE, DIM, _S_COLS), jnp.float32),
    )(table_t)

    mesh = plsc.VectorSubcoreMesh(core_axis_name="c", subcore_axis_name="s")
    out3 = pl.kernel(
        _copy_body,
        out_type=jax.ShapeDtypeStruct((LENGTH_Q, DIM, LENGTH_K), jnp.float32),
        mesh=mesh,
        scratch_types=[
            pltpu.VMEM((DIM, _S_COLS), jnp.float32),
            pltpu.SemaphoreType.DMA,
        ],
    )(st128)
    return jnp.transpose(out3, (0, 2, 1))


def kernel(length_q, length_k, rel_pos_embeddings):
    del length_q, length_k
    return _rpe(rel_pos_embeddings)
